# Initial kernel scaffold; baseline (speedup 1.0000x reference)
import jax
import jax.numpy as jnp
from jax.experimental import pallas as pl


def kernel(feat_p2, feat_p3, feat_p4, feat_p5, rois):
    raise NotImplementedError("write your pallas kernel here")


# speedup vs baseline: 14.5935x; 14.5935x over previous
import functools

import jax
import jax.numpy as jnp
from jax import lax
from jax.experimental import pallas as pl
from jax.experimental.pallas import tpu as pltpu
from jax.experimental.pallas import tpu_sc as plsc

C = 256
N_ROIS = 1000
SIZES = (256, 128, 64, 32)
TABLE_ROWS = 65536 + 16384 + 4096 + 1024
BINS = 49
TAPS = 16
CHUNK_BINS = 7
CHUNK_ROWS = CHUNK_BINS * TAPS
N_CHUNKS = BINS // CHUNK_BINS


def _sc_body(table, roisf, out, rois_v, yc_v, xc_v, wy_v, wx_v, idx_v, w_v,
             rows0, rows1, outb_v, sem0, sem1):
    cid = lax.axis_index("c")
    sid = lax.axis_index("s")
    wid = sid * 2 + cid
    base = jnp.where(wid < 8, wid * 32, 31 * wid + 8)
    count = jnp.where(wid < 8, 32, 31)

    pltpu.sync_copy(roisf, rois_v)

    iota = lax.iota(jnp.int32, 16)
    zeros_i = jnp.zeros((16,), jnp.int32)
    q = (iota >> 1).astype(jnp.float32) + (
        0.25 + 0.5 * (iota & 1).astype(jnp.float32))
    p_y = iota >> 2
    p_x = iota & 3
    ch_base = [(chn * 16 + iota) * BINS for chn in range(16)]

    def per_roi(i, _):
        g = base + i

        def splat(off):
            return plsc.load_gather(rois_v, [zeros_i + (g * 4 + off)])

        x1 = splat(0)
        y1 = splat(1)
        x2 = splat(2)
        y2 = splat(3)
        area = (y2 - y1) * (x2 - x1)
        k = ((area >= 12544.0).astype(jnp.int32)
             + (area >= 50176.0).astype(jnp.int32)
             + (area >= 200704.0).astype(jnp.int32))
        s_i = 256 >> k
        sf = s_i.astype(jnp.float32)
        scale = sf * (1.0 / 1024.0)
        boff = jnp.where(k == 0, 0,
                         jnp.where(k == 1, 65536,
                                   jnp.where(k == 2, 81920, 86016)))

        def axis_build(lo_img, hi_img, c_ref, w_ref):
            lo = lo_img * scale
            hi = hi_img * scale
            ln = jnp.maximum(hi - lo, 1.0)
            bsz = ln / 7.0
            gs = lo + q * bsz
            valid = (gs >= -1.0) & (gs <= sf)
            xx = jnp.maximum(gs, 0.0)
            fx = xx.astype(jnp.int32).astype(jnp.float32)
            clo = jnp.minimum(fx, sf - 1.0)
            xef = jnp.where(fx >= sf - 1.0, sf - 1.0, xx)
            chi = jnp.minimum(clo + 1.0, sf - 1.0)
            lw = xef - clo
            hw = 1.0 - lw
            plsc.store_scatter(c_ref, [2 * iota], clo.astype(jnp.int32))
            plsc.store_scatter(c_ref, [2 * iota + 1], chi.astype(jnp.int32))
            plsc.store_scatter(w_ref, [2 * iota], jnp.where(valid, hw, 0.0))
            plsc.store_scatter(w_ref, [2 * iota + 1], jnp.where(valid, lw, 0.0))

        axis_build(y1, y2, yc_v, wy_v)
        axis_build(x1, x2, xc_v, wx_v)

        def per_bin(b, _):
            oh = b // 7
            ow = b - oh * 7
            ysel = plsc.load_gather(yc_v, [p_y + 4 * oh])
            xsel = plsc.load_gather(xc_v, [p_x + 4 * ow])
            wys = plsc.load_gather(wy_v, [p_y + 4 * oh])
            wxs = plsc.load_gather(wx_v, [p_x + 4 * ow])
            idx16 = boff + ysel * s_i + xsel
            w16 = (0.25 * wys) * wxs
            plsc.store_scatter(idx_v, [b * 16 + iota], idx16)
            plsc.store_scatter(w_v, [b * 16 + iota], w16)
            return 0

        lax.fori_loop(0, BINS, per_bin, 0)

        bufs = (rows0, rows1)
        sems = (sem0, sem1)

        def fire(c):
            return pltpu.async_copy(
                table.at[idx_v.at[pl.ds(c * CHUNK_ROWS, CHUNK_ROWS)]],
                bufs[c % 2], sems[c % 2])

        cps = {0: fire(0), 1: fire(1)}
        for c in range(N_CHUNKS):
            cps[c].wait()
            rbuf = bufs[c % 2]

            def per_chunk_bin(bs, _):
                b = c * CHUNK_BINS + bs
                accs = [jnp.zeros((16,), jnp.float32) for _ in range(16)]
                for t in range(TAPS):
                    wsp = plsc.load_gather(w_v, [zeros_i + (b * 16 + t)])
                    r = bs * TAPS + t
                    for chn in range(16):
                        row = rbuf[r, pl.ds(chn * 16, 16)]
                        accs[chn] = accs[chn] + wsp * row
                for chn in range(16):
                    plsc.store_scatter(outb_v, [ch_base[chn] + b], accs[chn])
                return 0

            lax.fori_loop(0, CHUNK_BINS, per_chunk_bin, 0)
            if c + 2 < N_CHUNKS:
                cps[c + 2] = fire(c + 2)

        pltpu.sync_copy(outb_v, out.at[g])
        return 0

    lax.fori_loop(0, count, per_roi, 0)


@jax.jit
def _run(table, roisf):
    mesh = plsc.VectorSubcoreMesh(core_axis_name="c", subcore_axis_name="s")
    f = pl.kernel(
        _sc_body,
        out_type=jax.ShapeDtypeStruct((N_ROIS, C * BINS), jnp.float32),
        mesh=mesh,
        scratch_types=[
            pltpu.VMEM((N_ROIS * 4,), jnp.float32),
            pltpu.VMEM((32,), jnp.int32),
            pltpu.VMEM((32,), jnp.int32),
            pltpu.VMEM((32,), jnp.float32),
            pltpu.VMEM((32,), jnp.float32),
            pltpu.VMEM((BINS * TAPS,), jnp.int32),
            pltpu.VMEM((BINS * TAPS,), jnp.float32),
            pltpu.VMEM((CHUNK_ROWS, C), jnp.float32),
            pltpu.VMEM((CHUNK_ROWS, C), jnp.float32),
            pltpu.VMEM((C * BINS,), jnp.float32),
            pltpu.SemaphoreType.DMA,
            pltpu.SemaphoreType.DMA,
        ],
        compiler_params=pltpu.CompilerParams(needs_layout_passes=False),
    )
    return f(table, roisf)


def kernel(feat_p2, feat_p3, feat_p4, feat_p5, rois):
    tabs = []
    for f in (feat_p2, feat_p3, feat_p4, feat_p5):
        s = f.shape[-1]
        tabs.append(jnp.transpose(f[0], (1, 2, 0)).reshape(s * s, C))
    table = jnp.concatenate(tabs, axis=0)
    out = _run(table, rois.reshape(-1))
    return out.reshape(N_ROIS, C, 7, 7)
